# baseline (speedup 1.0000x reference)
import jax
import jax.numpy as jnp
from jax import lax
from jax.experimental import pallas as pl
from jax.experimental.pallas import tpu as pltpu
from jax.experimental.pallas import tpu_sc as plsc

R = 128
N = 32768
L = 16
NC = 2
NS = 16
NW = NC * NS
RPW = R // NW
NCH = N // L
NBIS = 25
PAD = 1e-6
BIG = 3.0e38

_mesh = plsc.VectorSubcoreMesh(
    core_axis_name="c", subcore_axis_name="s",
    num_cores=NC, num_subcores=NS)


def _body(x_hbm, out_hbm, row_v, cand_v, pcs_v, offs_v):
    wid = lax.axis_index("s") * NC + lax.axis_index("c")
    zeros = jnp.zeros((L,), jnp.float32)

    def bcast(s):
        return jnp.broadcast_to(s, (L,))

    def scal(v):
        return lax.squeeze(lax.slice(v, (0,), (1,)), (0,))

    def scal_last(v):
        return lax.squeeze(lax.slice(v, (L - 1,), (L,)), (0,))

    def final_tau(o, o_end, nch_c):
        def body(i, carry):
            khi, shi, cin, sin, mn, mx = carry
            v = cand_v[pl.ds(i * L, L)]
            rel = v - o
            m_hi = v >= o_end
            m_in = (v > o) & (v < o_end)
            khi = khi + jnp.where(m_hi, 1.0, 0.0)
            shi = shi + jnp.where(m_hi, rel, 0.0)
            cin = cin + jnp.where(m_in, 1.0, 0.0)
            sin = sin + jnp.where(m_in, rel, 0.0)
            mn = jnp.minimum(mn, jnp.where(m_in, rel, BIG))
            mx = jnp.maximum(mx, jnp.where(m_in, rel, -BIG))
            return khi, shi, cin, sin, mn, mx

        init = (zeros, zeros, zeros, zeros,
                jnp.full((L,), BIG, jnp.float32),
                jnp.full((L,), -BIG, jnp.float32))
        khi, shi, cin, sin, mn, mx = lax.fori_loop(0, nch_c, body, init)
        k_hi = bcast(jnp.sum(khi))
        s_hi = bcast(jnp.sum(shi))
        c_in = bcast(jnp.sum(cin))
        s_in = bcast(jnp.sum(sin))
        minr = bcast(jnp.min(mn))
        maxr = bcast(jnp.max(mx))
        d_a = (s_hi - 1.0) / jnp.maximum(k_hi, 1.0)
        d_b = (s_hi + s_in - 1.0) / jnp.maximum(k_hi + c_in, 1.0)
        d_c = (s_hi + maxr - 1.0) / (k_hi + 1.0)
        valid_a = (k_hi > 0.0) & ((c_in == 0.0) | (d_a >= maxr))
        valid_b = ((k_hi + c_in) > 0.0) & ((c_in == 0.0) | (d_b < minr))
        valid_c = (c_in > 0.0) & (d_c < maxr) & ((c_in == 1.0) | (d_c >= minr))
        d = jnp.where(valid_a, d_a,
                      jnp.where(valid_b, d_b,
                                jnp.where(valid_c, d_c, d_b)))
        return o + d

    def row_body(rr, carry):
        r = wid * RPW + rr
        pltpu.sync_copy(x_hbm.at[r], row_v)

        def max_body(i, accs):
            a0, a1, a2, a3 = accs
            a0 = jnp.maximum(a0, row_v[pl.ds((4 * i) * L, L)])
            a1 = jnp.maximum(a1, row_v[pl.ds((4 * i + 1) * L, L)])
            a2 = jnp.maximum(a2, row_v[pl.ds((4 * i + 2) * L, L)])
            a3 = jnp.maximum(a3, row_v[pl.ds((4 * i + 3) * L, L)])
            return a0, a1, a2, a3

        neg = jnp.full((L,), -BIG, jnp.float32)
        a0, a1, a2, a3 = lax.fori_loop(0, NCH // 4, max_body,
                                       (neg, neg, neg, neg), unroll=4)
        m = bcast(jnp.max(jnp.maximum(jnp.maximum(a0, a1),
                                      jnp.maximum(a2, a3))))
        start = m - 1.001

        def pc_body(i, carry):
            msk = row_v[pl.ds(i * L, L)] > start
            pcs_v[pl.ds(i * L, L)] = plsc.all_reduce_population_count(msk)
            return carry

        lax.fori_loop(0, NCH, pc_body, 0, unroll=8)
        iota_i = lax.iota(jnp.int32, L)

        def pfx_body(t, carry):
            idx = (t * L + iota_i) * L
            pv = plsc.load_gather(pcs_v, [idx])
            inc = jnp.cumsum(pv)
            offs_v[pl.ds(t * L, L)] = inc - pv + bcast(carry)
            return carry + scal_last(inc)

        off = lax.fori_loop(0, NCH // L, pfx_body, jnp.zeros((), jnp.int32))

        def store_body(i, carry):
            v = row_v[pl.ds(i * L, L)]
            msk = v > start
            off_i = scal(offs_v[pl.ds(i, L)])
            plsc.store_compressed(cand_v.at[pl.ds(off_i, L)], v, mask=msk)
            return carry

        lax.fori_loop(0, NCH, store_body, 0, unroll=8)
        cand_v[pl.ds(off, L)] = jnp.full((L,), -BIG, jnp.float32)
        nch_c = lax.shift_right_logical(off + (L - 1), 4)

        def bis_body(_, carry):
            lo, hi = carry
            mid = 0.5 * (lo + hi)

            def acc_body(i, acc):
                v = cand_v[pl.ds(i * L, L)]
                return acc + jnp.where(v > mid, v - mid, 0.0)

            f = bcast(jnp.sum(lax.fori_loop(0, nch_c, acc_body, zeros)))
            gt = f > 1.0
            return jnp.where(gt, mid, lo), jnp.where(gt, hi, mid)

        lo, hi = lax.fori_loop(0, NBIS, bis_body,
                               (start, m + 0.001))
        tau = final_tau(lo - PAD, hi + PAD, nch_c)

        def out_body(i, c):
            v = row_v[pl.ds(i * L, L)]
            row_v[pl.ds(i * L, L)] = jnp.maximum(v - tau, 0.0)
            return c

        lax.fori_loop(0, NCH, out_body, 0, unroll=8)
        pltpu.sync_copy(row_v, out_hbm.at[r])
        return carry

    lax.fori_loop(0, RPW, row_body, 0)


_sparsemax_sc = pl.kernel(
    _body,
    out_type=jax.ShapeDtypeStruct((R, N), jnp.float32),
    mesh=_mesh,
    compiler_params=pltpu.CompilerParams(needs_layout_passes=False),
    scratch_types=[
        pltpu.VMEM((N,), jnp.float32),
        pltpu.VMEM((N + L,), jnp.float32),
        pltpu.VMEM((N,), jnp.int32),
        pltpu.VMEM((NCH + L,), jnp.int32),
    ],
)


@jax.jit
def kernel(logits):
    return _sparsemax_sc(logits)


# speedup vs baseline: 2.3948x; 2.3948x over previous
import jax
import jax.numpy as jnp
from jax import lax
from jax.experimental import pallas as pl
from jax.experimental.pallas import tpu as pltpu
from jax.experimental.pallas import tpu_sc as plsc

R = 128
N = 32768
L = 16
NC = 2
NS = 16
NW = NC * NS
RPW = R // NW
NCH = N // L
NBIS = 25
PAD = 1e-6
BIG = 3.0e38

_mesh = plsc.VectorSubcoreMesh(
    core_axis_name="c", subcore_axis_name="s",
    num_cores=NC, num_subcores=NS)


def _body(x_hbm, out_hbm, row_v, cand_v, pcs_v, offs_v):
    wid = lax.axis_index("s") * NC + lax.axis_index("c")
    zeros = jnp.zeros((L,), jnp.float32)

    def bcast(s):
        return jnp.broadcast_to(s, (L,))

    def scal(v):
        return lax.squeeze(lax.slice(v, (0,), (1,)), (0,))

    def scal_last(v):
        return lax.squeeze(lax.slice(v, (L - 1,), (L,)), (0,))

    def final_tau(o, o_end, nch_c):
        def body(i, carry):
            khi, shi, cin, sin, mn, mx = carry
            v = cand_v[pl.ds(i * L, L)]
            rel = v - o
            m_hi = v >= o_end
            m_in = (v > o) & (v < o_end)
            khi = khi + jnp.where(m_hi, 1.0, 0.0)
            shi = shi + jnp.where(m_hi, rel, 0.0)
            cin = cin + jnp.where(m_in, 1.0, 0.0)
            sin = sin + jnp.where(m_in, rel, 0.0)
            mn = jnp.minimum(mn, jnp.where(m_in, rel, BIG))
            mx = jnp.maximum(mx, jnp.where(m_in, rel, -BIG))
            return khi, shi, cin, sin, mn, mx

        init = (zeros, zeros, zeros, zeros,
                jnp.full((L,), BIG, jnp.float32),
                jnp.full((L,), -BIG, jnp.float32))
        khi, shi, cin, sin, mn, mx = plsc.parallel_loop(
            0, nch_c, carry=init)(body)
        k_hi = bcast(jnp.sum(khi))
        s_hi = bcast(jnp.sum(shi))
        c_in = bcast(jnp.sum(cin))
        s_in = bcast(jnp.sum(sin))
        minr = bcast(jnp.min(mn))
        maxr = bcast(jnp.max(mx))
        d_a = (s_hi - 1.0) / jnp.maximum(k_hi, 1.0)
        d_b = (s_hi + s_in - 1.0) / jnp.maximum(k_hi + c_in, 1.0)
        d_c = (s_hi + maxr - 1.0) / (k_hi + 1.0)
        valid_a = (k_hi > 0.0) & ((c_in == 0.0) | (d_a >= maxr))
        valid_b = ((k_hi + c_in) > 0.0) & ((c_in == 0.0) | (d_b < minr))
        valid_c = (c_in > 0.0) & (d_c < maxr) & ((c_in == 1.0) | (d_c >= minr))
        d = jnp.where(valid_a, d_a,
                      jnp.where(valid_b, d_b,
                                jnp.where(valid_c, d_c, d_b)))
        return o + d

    def row_body(rr, carry):
        r = wid * RPW + rr
        pltpu.sync_copy(x_hbm.at[r], row_v)

        def max_body(i, accs):
            a0, a1, a2, a3 = accs
            a0 = jnp.maximum(a0, row_v[pl.ds((4 * i) * L, L)])
            a1 = jnp.maximum(a1, row_v[pl.ds((4 * i + 1) * L, L)])
            a2 = jnp.maximum(a2, row_v[pl.ds((4 * i + 2) * L, L)])
            a3 = jnp.maximum(a3, row_v[pl.ds((4 * i + 3) * L, L)])
            return a0, a1, a2, a3

        neg = jnp.full((L,), -BIG, jnp.float32)
        a0, a1, a2, a3 = plsc.parallel_loop(
            0, NCH // 4, unroll=4, carry=(neg, neg, neg, neg))(max_body)
        m = bcast(jnp.max(jnp.maximum(jnp.maximum(a0, a1),
                                      jnp.maximum(a2, a3))))
        start = m - 1.001

        def pc_body(i):
            msk = row_v[pl.ds(i * L, L)] > start
            pcs_v[pl.ds(i * L, L)] = plsc.all_reduce_population_count(msk)

        plsc.parallel_loop(0, NCH, unroll=8)(pc_body)
        iota_i = lax.iota(jnp.int32, L)

        def pfx_body(t, carry):
            idx = (t * L + iota_i) * L
            pv = plsc.load_gather(pcs_v, [idx])
            inc = jnp.cumsum(pv)
            offs_v[pl.ds(t * L, L)] = inc - pv + bcast(carry)
            return carry + scal_last(inc)

        off = lax.fori_loop(0, NCH // L, pfx_body, jnp.zeros((), jnp.int32))

        def store_body(i):
            v = row_v[pl.ds(i * L, L)]
            msk = v > start
            off_i = scal(offs_v[pl.ds(i, L)])
            plsc.store_compressed(cand_v.at[pl.ds(off_i, L)], v, mask=msk)

        plsc.parallel_loop(0, NCH, unroll=8)(store_body)
        cand_v[pl.ds(off, L)] = jnp.full((L,), -BIG, jnp.float32)
        nch_c = lax.shift_right_logical(off + (L - 1), 4)

        def bis_body(_, carry):
            lo, hi = carry
            mid = 0.5 * (lo + hi)

            def acc_body(i, acc):
                v = cand_v[pl.ds(i * L, L)]
                return acc + jnp.where(v > mid, v - mid, 0.0)

            f = bcast(jnp.sum(
                plsc.parallel_loop(0, nch_c, carry=zeros)(acc_body)))
            gt = f > 1.0
            return jnp.where(gt, mid, lo), jnp.where(gt, hi, mid)

        lo, hi = lax.fori_loop(0, NBIS, bis_body,
                               (start, m + 0.001))
        tau = final_tau(lo - PAD, hi + PAD, nch_c)

        def out_body(i):
            v = row_v[pl.ds(i * L, L)]
            row_v[pl.ds(i * L, L)] = jnp.maximum(v - tau, 0.0)

        plsc.parallel_loop(0, NCH, unroll=8)(out_body)
        pltpu.sync_copy(row_v, out_hbm.at[r])
        return carry

    lax.fori_loop(0, RPW, row_body, 0)


_sparsemax_sc = pl.kernel(
    _body,
    out_type=jax.ShapeDtypeStruct((R, N), jnp.float32),
    mesh=_mesh,
    compiler_params=pltpu.CompilerParams(needs_layout_passes=False),
    scratch_types=[
        pltpu.VMEM((N,), jnp.float32),
        pltpu.VMEM((N + L,), jnp.float32),
        pltpu.VMEM((N,), jnp.int32),
        pltpu.VMEM((NCH + L,), jnp.int32),
    ],
)


@jax.jit
def kernel(logits):
    return _sparsemax_sc(logits)


# speedup vs baseline: 2.7588x; 1.1520x over previous
import jax
import jax.numpy as jnp
from jax import lax
from jax.experimental import pallas as pl
from jax.experimental.pallas import tpu as pltpu
from jax.experimental.pallas import tpu_sc as plsc

R = 128
N = 32768
L = 16
NC = 2
NS = 16
NW = NC * NS
RPW = R // NW
NCH = N // L
NBIS = 25
PAD = 1e-6
BIG = 3.0e38

_mesh = plsc.VectorSubcoreMesh(
    core_axis_name="c", subcore_axis_name="s",
    num_cores=NC, num_subcores=NS)


def _body(x_hbm, out_hbm, rowa_v, rowb_v, cand_v, pcs_v, offs_v, sema, semb, semo):
    wid = lax.axis_index("s") * NC + lax.axis_index("c")
    zeros = jnp.zeros((L,), jnp.float32)

    def bcast(s):
        return jnp.broadcast_to(s, (L,))

    def scal(v):
        return lax.squeeze(lax.slice(v, (0,), (1,)), (0,))

    def scal_last(v):
        return lax.squeeze(lax.slice(v, (L - 1,), (L,)), (0,))

    def final_tau(o, o_end, nch_c):
        def body(i, carry):
            khi, shi, cin, sin, mn, mx = carry
            v = cand_v[pl.ds(i * L, L)]
            rel = v - o
            m_hi = v >= o_end
            m_in = (v > o) & (v < o_end)
            khi = khi + jnp.where(m_hi, 1.0, 0.0)
            shi = shi + jnp.where(m_hi, rel, 0.0)
            cin = cin + jnp.where(m_in, 1.0, 0.0)
            sin = sin + jnp.where(m_in, rel, 0.0)
            mn = jnp.minimum(mn, jnp.where(m_in, rel, BIG))
            mx = jnp.maximum(mx, jnp.where(m_in, rel, -BIG))
            return khi, shi, cin, sin, mn, mx

        init = (zeros, zeros, zeros, zeros,
                jnp.full((L,), BIG, jnp.float32),
                jnp.full((L,), -BIG, jnp.float32))
        khi, shi, cin, sin, mn, mx = plsc.parallel_loop(
            0, nch_c, carry=init)(body)
        k_hi = bcast(jnp.sum(khi))
        s_hi = bcast(jnp.sum(shi))
        c_in = bcast(jnp.sum(cin))
        s_in = bcast(jnp.sum(sin))
        minr = bcast(jnp.min(mn))
        maxr = bcast(jnp.max(mx))
        d_a = (s_hi - 1.0) / jnp.maximum(k_hi, 1.0)
        d_b = (s_hi + s_in - 1.0) / jnp.maximum(k_hi + c_in, 1.0)
        d_c = (s_hi + maxr - 1.0) / (k_hi + 1.0)
        valid_a = (k_hi > 0.0) & ((c_in == 0.0) | (d_a >= maxr))
        valid_b = ((k_hi + c_in) > 0.0) & ((c_in == 0.0) | (d_b < minr))
        valid_c = (c_in > 0.0) & (d_c < maxr) & ((c_in == 1.0) | (d_c >= minr))
        d = jnp.where(valid_a, d_a,
                      jnp.where(valid_b, d_b,
                                jnp.where(valid_c, d_c, d_b)))
        return o + d

    def compute_pre(buf):
        def max_body(i, accs):
            a0, a1, a2, a3 = accs
            a0 = jnp.maximum(a0, buf[pl.ds((4 * i) * L, L)])
            a1 = jnp.maximum(a1, buf[pl.ds((4 * i + 1) * L, L)])
            a2 = jnp.maximum(a2, buf[pl.ds((4 * i + 2) * L, L)])
            a3 = jnp.maximum(a3, buf[pl.ds((4 * i + 3) * L, L)])
            return a0, a1, a2, a3

        neg = jnp.full((L,), -BIG, jnp.float32)
        a0, a1, a2, a3 = plsc.parallel_loop(
            0, NCH // 4, unroll=4, carry=(neg, neg, neg, neg))(max_body)
        m = bcast(jnp.max(jnp.maximum(jnp.maximum(a0, a1),
                                      jnp.maximum(a2, a3))))
        start = m - 1.001
        lane0 = lax.iota(jnp.int32, L) == 0

        def pc_body(i):
            msk = buf[pl.ds(i * L, L)] > start
            plsc.store_compressed(pcs_v.at[pl.ds(i, L)],
                                  plsc.all_reduce_population_count(msk),
                                  mask=lane0)

        plsc.parallel_loop(0, NCH, unroll=8)(pc_body)
        return m, start

    def compact(buf, start):
        def pfx_body(t, carry):
            pv = pcs_v[pl.ds(t * L, L)]
            inc = jnp.cumsum(pv)
            offs_v[pl.ds(t * L, L)] = inc - pv + bcast(carry)
            return carry + scal_last(inc)

        off = lax.fori_loop(0, NCH // L, pfx_body, jnp.zeros((), jnp.int32))

        def store_body(i):
            v = buf[pl.ds(i * L, L)]
            msk = v > start
            off_i = scal(offs_v[pl.ds(i, L)])
            plsc.store_compressed(cand_v.at[pl.ds(off_i, L)], v, mask=msk)

        plsc.parallel_loop(0, NCH, unroll=8)(store_body)
        cand_v[pl.ds(off, L)] = jnp.full((L,), -BIG, jnp.float32)
        return lax.shift_right_logical(off + (L - 1), 4)

    def bisect(m, start, nch_c):
        def bis_body(_, carry):
            lo, hi = carry
            mid = 0.5 * (lo + hi)

            def acc_body(i, acc):
                v = cand_v[pl.ds(i * L, L)]
                return acc + jnp.where(v > mid, v - mid, 0.0)

            f = bcast(jnp.sum(
                plsc.parallel_loop(0, nch_c, carry=zeros)(acc_body)))
            gt = f > 1.0
            return jnp.where(gt, mid, lo), jnp.where(gt, hi, mid)

        return lax.fori_loop(0, NBIS, bis_body, (start, m + 0.001))

    def out_pass(buf, tau):
        def out_body(i):
            v = buf[pl.ds(i * L, L)]
            cand_v[pl.ds(i * L, L)] = jnp.maximum(v - tau, 0.0)

        plsc.parallel_loop(0, NCH, unroll=8)(out_body)

    bufs = [rowa_v, rowb_v]
    sems = [sema, semb]
    r0 = wid * RPW
    h_in = [None] * RPW
    h_in[0] = pltpu.async_copy(x_hbm.at[r0], bufs[0], sems[0])
    h_in[1] = pltpu.async_copy(x_hbm.at[r0 + 1], bufs[1], sems[1])
    h_out = None
    for rr in range(RPW):
        buf = bufs[rr % 2]
        h_in[rr].wait()
        m, start = compute_pre(buf)
        if h_out is not None:
            h_out.wait()
        nch_c = compact(buf, start)
        lo, hi = bisect(m, start, nch_c)
        tau = final_tau(lo - PAD, hi + PAD, nch_c)
        out_pass(buf, tau)
        h_out = pltpu.async_copy(cand_v.at[pl.ds(0, N)],
                                 out_hbm.at[r0 + rr], semo)
        if rr + 2 < RPW:
            h_in[rr + 2] = pltpu.async_copy(x_hbm.at[r0 + rr + 2], buf,
                                            sems[rr % 2])
    h_out.wait()


_sparsemax_sc = pl.kernel(
    _body,
    out_type=jax.ShapeDtypeStruct((R, N), jnp.float32),
    mesh=_mesh,
    compiler_params=pltpu.CompilerParams(needs_layout_passes=False),
    scratch_types=[
        pltpu.VMEM((N,), jnp.float32),
        pltpu.VMEM((N,), jnp.float32),
        pltpu.VMEM((N + L,), jnp.float32),
        pltpu.VMEM((NCH + L,), jnp.int32),
        pltpu.VMEM((NCH + L,), jnp.int32),
        pltpu.SemaphoreType.DMA,
        pltpu.SemaphoreType.DMA,
        pltpu.SemaphoreType.DMA,
    ],
)


@jax.jit
def kernel(logits):
    return _sparsemax_sc(logits)
